# speedup vs baseline: 1.1007x; 1.1007x over previous
import functools

import jax
import jax.numpy as jnp
from jax.experimental import pallas as pl
from jax.experimental.pallas import tpu as pltpu

N = 10000
E = 320000
D = 128
H = 128
EPS = 1e-7
MAX_LOGSTD = 10.0
BN_EPS = 1e-5

ROW_BLK = 1000


def _linear_body(x_ref, w_ref, b_ref, o_ref):
    o_ref[...] = (
        jnp.dot(x_ref[...], w_ref[...], preferred_element_type=jnp.float32)
        + b_ref[...]
    )


@jax.jit
def _linear(x, w, b):
    return pl.pallas_call(
        _linear_body,
        grid=(N // ROW_BLK,),
        in_specs=[
            pl.BlockSpec((ROW_BLK, D), lambda i: (i, 0)),
            pl.BlockSpec((D, H), lambda i: (0, 0)),
            pl.BlockSpec((1, H), lambda i: (0, 0)),
        ],
        out_specs=pl.BlockSpec((ROW_BLK, H), lambda i: (i, 0)),
        out_shape=jax.ShapeDtypeStruct((N, H), jnp.float32),
    )(x, w, b.reshape(1, H))


def _mlp_body(mode, agg_ref, h_ref, w1_ref, g1_ref, b1_ref, w2_ref, o_ref):
    out = agg_ref[...] + h_ref[...]
    hh = jnp.dot(out, w1_ref[...], preferred_element_type=jnp.float32)
    hh = hh * (1.0 / jnp.sqrt(1.0 + BN_EPS)) * g1_ref[...] + b1_ref[...]
    hh = jnp.maximum(hh, 0.0)
    o = jnp.dot(hh, w2_ref[...], preferred_element_type=jnp.float32)
    if mode == "relu":
        o = jnp.maximum(o, 0.0)
    elif mode == "clamp":
        o = jnp.minimum(o, MAX_LOGSTD)
    o_ref[...] = o


@functools.partial(jax.jit, static_argnames=("mode",))
def _mlp(agg, h, w1, g1, b1, w2, mode):
    return pl.pallas_call(
        functools.partial(_mlp_body, mode),
        grid=(N // ROW_BLK,),
        in_specs=[
            pl.BlockSpec((ROW_BLK, H), lambda i: (i, 0)),
            pl.BlockSpec((ROW_BLK, H), lambda i: (i, 0)),
            pl.BlockSpec((H, 2 * H), lambda i: (0, 0)),
            pl.BlockSpec((1, 2 * H), lambda i: (0, 0)),
            pl.BlockSpec((1, 2 * H), lambda i: (0, 0)),
            pl.BlockSpec((2 * H, H), lambda i: (0, 0)),
        ],
        out_specs=pl.BlockSpec((ROW_BLK, H), lambda i: (i, 0)),
        out_shape=jax.ShapeDtypeStruct((N, H), jnp.float32),
    )(agg, h, w1, g1.reshape(1, 2 * H), b1.reshape(1, 2 * H), w2)


def _reparam_body(mu_ref, ls_ref, nz_ref, o_ref):
    o_ref[...] = mu_ref[...] + nz_ref[...] * jnp.exp(ls_ref[...])


@jax.jit
def _reparam(mu, logstd, noise):
    return pl.pallas_call(
        _reparam_body,
        grid=(N // ROW_BLK,),
        in_specs=[pl.BlockSpec((ROW_BLK, H), lambda i: (i, 0))] * 3,
        out_specs=pl.BlockSpec((ROW_BLK, H), lambda i: (i, 0)),
        out_shape=jax.ShapeDtypeStruct((N, H), jnp.float32),
    )(mu, logstd, noise)


def _aggregate(h, src, dst, t):
    msg = jnp.maximum(h[src], 0.0) + EPS
    alpha = msg * t
    seg_max = jax.ops.segment_max(alpha, dst, num_segments=N)
    alpha = jnp.exp(alpha - seg_max[dst])
    denom = jax.ops.segment_sum(alpha, dst, num_segments=N)
    w = alpha / (denom[dst] + 1e-16)
    return jax.ops.segment_sum(msg * w, dst, num_segments=N)


def kernel(x, edge_index, Wn, bn, t, W1, g1, b1, W2, noise):
    src = edge_index[0]
    dst = edge_index[1]
    h = _linear(x, Wn, bn)
    for i in range(2):
        agg = _aggregate(h, src, dst, t[i])
        h = _mlp(agg, h, W1[i], g1[i], b1[i], W2[i], "relu")
    agg = _aggregate(h, src, dst, t[2])
    mu = _mlp(agg, h, W1[2], g1[2], b1[2], W2[2], "none")
    agg = _aggregate(h, src, dst, t[3])
    logstd = _mlp(agg, h, W1[3], g1[3], b1[3], W2[3], "clamp")
    zh = Z = _reparam(mu, logstd, noise)
    for i in (4, 5):
        agg = _aggregate(zh, src, dst, t[i])
        zh = _mlp(agg, zh, W1[i], g1[i], b1[i], W2[i], "relu")
    return (zh, Z)


# speedup vs baseline: 6.5157x; 5.9195x over previous
import functools

import jax
import jax.numpy as jnp
from jax import lax
from jax.experimental import pallas as pl
from jax.experimental.pallas import tpu as pltpu
from jax.experimental.pallas import tpu_sc as plsc

N = 10000
E = 320000
D = 128
H = 128
EPS = 1e-7
MAX_LOGSTD = 10.0
BN_EPS = 1e-5

NW = 32
NPT = 313
N2 = NW * NPT
BE = 128
ROW_BLK = 2504
NEG = -1e30

_mesh = plsc.VectorSubcoreMesh(core_axis_name="c", subcore_axis_name="s")


def _agg_body(h_hbm, src_hbm, dst_hbm, meta_hbm, t_hbm, out_hbm,
              stage, rows, srcw, dstw, mrow_v, tv):
    wid = lax.axis_index("c") * 16 + lax.axis_index("s")
    pltpu.sync_copy(meta_hbm.at[wid], mrow_v)
    pltpu.sync_copy(t_hbm, tv)
    mv = mrow_v[...]
    e0 = mv[0]
    e1 = mv[1]
    n0 = mv[2]

    @pl.loop(0, NPT * H, step=16)
    def _(i):
        stage[pl.ds(i, 16)] = jnp.zeros((16,), jnp.float32)

    t_vec = tv[...]

    def edge_body(j, carry):
        cur, ms, ss, ns = carry
        d = dstw[pl.ds(j, 16)][0]
        changed = d != cur

        @pl.when(jnp.logical_and(changed, cur >= 0))
        def _():
            r = cur - n0
            for k in range(8):
                stage[pl.ds(r * H + 16 * k, 16)] = ns[k] / ss[k]

        new_m, new_s, new_n = [], [], []
        for k in range(8):
            row = rows[j, pl.ds(16 * k, 16)]
            msg = jnp.maximum(row, 0.0) + EPS
            alpha = msg * t_vec
            m_eff = jnp.where(changed, NEG, ms[k])
            ge = alpha >= m_eff
            q = jnp.exp(-jnp.abs(alpha - m_eff))
            scale = jnp.where(ge, q, 1.0)
            w = jnp.where(ge, 1.0, q)
            new_m.append(jnp.maximum(m_eff, alpha))
            new_s.append(ss[k] * scale + w)
            new_n.append(ns[k] * scale + msg * w)
        return (d, tuple(new_m), tuple(new_s), tuple(new_n))

    def batch_body(bb, carry):
        base = bb * BE
        pltpu.sync_copy(src_hbm.at[pl.ds(base, BE)], srcw)
        pltpu.sync_copy(dst_hbm.at[pl.ds(base, BE)], dstw.at[pl.ds(0, BE)])
        pltpu.sync_copy(h_hbm.at[srcw], rows)
        jstart = jnp.maximum(e0 - base, 0)
        jend = jnp.minimum(e1 - base, BE)
        return lax.fori_loop(jstart, jend, edge_body, carry)

    init = (jnp.int32(-1),
            tuple(jnp.full((16,), NEG, jnp.float32) for _ in range(8)),
            tuple(jnp.zeros((16,), jnp.float32) for _ in range(8)),
            tuple(jnp.zeros((16,), jnp.float32) for _ in range(8)))
    bb0 = e0 // BE
    bb1 = (e1 + BE - 1) // BE
    cur, ms, ss, ns = lax.fori_loop(bb0, bb1, batch_body, init)

    @pl.when(cur >= 0)
    def _():
        r = cur - n0
        for k in range(8):
            stage[pl.ds(r * H + 16 * k, 16)] = ns[k] / ss[k]

    pltpu.sync_copy(stage, out_hbm.at[pl.ds(n0 * H, NPT * H)])


@jax.jit
def _sc_aggregate(h, src_s, dst_s, meta, t16):
    k = pl.kernel(
        _agg_body,
        out_type=jax.ShapeDtypeStruct((N2 * H,), jnp.float32),
        mesh=_mesh,
        scratch_types=[
            pltpu.VMEM((NPT * H,), jnp.float32),
            pltpu.VMEM((BE, H), jnp.float32),
            pltpu.VMEM((BE,), jnp.int32),
            pltpu.VMEM((BE + 16,), jnp.int32),
            pltpu.VMEM((16,), jnp.int32),
            pltpu.VMEM((16,), jnp.float32),
        ],
    )
    return k(h, src_s, dst_s, meta, t16).reshape(N2, H)


def _linear_body(x_ref, w_ref, b_ref, o_ref):
    o_ref[...] = (
        jnp.dot(x_ref[...], w_ref[...], preferred_element_type=jnp.float32)
        + b_ref[...]
    )


@jax.jit
def _linear(x, w, b):
    return pl.pallas_call(
        _linear_body,
        grid=(N2 // ROW_BLK,),
        in_specs=[
            pl.BlockSpec((ROW_BLK, D), lambda i: (i, 0)),
            pl.BlockSpec((D, H), lambda i: (0, 0)),
            pl.BlockSpec((1, H), lambda i: (0, 0)),
        ],
        out_specs=pl.BlockSpec((ROW_BLK, H), lambda i: (i, 0)),
        out_shape=jax.ShapeDtypeStruct((N2, H), jnp.float32),
    )(x, w, b.reshape(1, H))


def _mlp_body(mode, agg_ref, h_ref, w1_ref, g1_ref, b1_ref, w2_ref, o_ref):
    out = agg_ref[...] + h_ref[...]
    hh = jnp.dot(out, w1_ref[...], preferred_element_type=jnp.float32)
    hh = hh * (1.0 / jnp.sqrt(1.0 + BN_EPS)) * g1_ref[...] + b1_ref[...]
    hh = jnp.maximum(hh, 0.0)
    o = jnp.dot(hh, w2_ref[...], preferred_element_type=jnp.float32)
    if mode == "relu":
        o = jnp.maximum(o, 0.0)
    elif mode == "clamp":
        o = jnp.minimum(o, MAX_LOGSTD)
    o_ref[...] = o


@functools.partial(jax.jit, static_argnames=("mode",))
def _mlp(agg, h, w1, g1, b1, w2, mode):
    return pl.pallas_call(
        functools.partial(_mlp_body, mode),
        grid=(N2 // ROW_BLK,),
        in_specs=[
            pl.BlockSpec((ROW_BLK, H), lambda i: (i, 0)),
            pl.BlockSpec((ROW_BLK, H), lambda i: (i, 0)),
            pl.BlockSpec((H, 2 * H), lambda i: (0, 0)),
            pl.BlockSpec((1, 2 * H), lambda i: (0, 0)),
            pl.BlockSpec((1, 2 * H), lambda i: (0, 0)),
            pl.BlockSpec((2 * H, H), lambda i: (0, 0)),
        ],
        out_specs=pl.BlockSpec((ROW_BLK, H), lambda i: (i, 0)),
        out_shape=jax.ShapeDtypeStruct((N2, H), jnp.float32),
    )(agg, h, w1, g1.reshape(1, 2 * H), b1.reshape(1, 2 * H), w2)


def _reparam_body(mu_ref, ls_ref, nz_ref, o_ref):
    o_ref[...] = mu_ref[...] + nz_ref[...] * jnp.exp(ls_ref[...])


@jax.jit
def _reparam(mu, logstd, noise):
    return pl.pallas_call(
        _reparam_body,
        grid=(N2 // ROW_BLK,),
        in_specs=[pl.BlockSpec((ROW_BLK, H), lambda i: (i, 0))] * 3,
        out_specs=pl.BlockSpec((ROW_BLK, H), lambda i: (i, 0)),
        out_shape=jax.ShapeDtypeStruct((N2, H), jnp.float32),
    )(mu, logstd, noise)


def kernel(x, edge_index, Wn, bn, t, W1, g1, b1, W2, noise):
    src = edge_index[0].astype(jnp.int32)
    dst = edge_index[1].astype(jnp.int32)

    perm = jnp.argsort(dst)
    dst_s = dst[perm]
    src_s = src[perm]
    tile_nodes = jnp.arange(NW + 1, dtype=jnp.int32) * NPT
    bounds = jnp.searchsorted(dst_s, tile_nodes).astype(jnp.int32)
    meta = jnp.concatenate(
        [bounds[:-1, None], bounds[1:, None], tile_nodes[:-1, None],
         jnp.zeros((NW, 13), jnp.int32)], axis=1)

    x_p = jnp.pad(x, ((0, N2 - N), (0, 0)))
    noise_p = jnp.pad(noise, ((0, N2 - N), (0, 0)))

    def agg(h, i):
        return _sc_aggregate(h, src_s, dst_s, meta, jnp.full((16,), t[i]))

    h = _linear(x_p, Wn, bn)
    for i in range(2):
        h = _mlp(agg(h, i), h, W1[i], g1[i], b1[i], W2[i], "relu")
    mu = _mlp(agg(h, 2), h, W1[2], g1[2], b1[2], W2[2], "none")
    logstd = _mlp(agg(h, 3), h, W1[3], g1[3], b1[3], W2[3], "clamp")
    zh = Z = _reparam(mu, logstd, noise_p)
    for i in (4, 5):
        zh = _mlp(agg(zh, i), zh, W1[i], g1[i], b1[i], W2[i], "relu")
    return (zh[:N], Z[:N])


# speedup vs baseline: 6.8686x; 1.0542x over previous
import functools

import jax
import jax.numpy as jnp
from jax import lax
from jax.experimental import pallas as pl
from jax.experimental.pallas import tpu as pltpu
from jax.experimental.pallas import tpu_sc as plsc

N = 10000
E = 320000
D = 128
H = 128
EPS = 1e-7
MAX_LOGSTD = 10.0
BN_EPS = 1e-5

NW = 32
NPT = 313
N2 = NW * NPT
BE = 128
ROW_BLK = 2504
NEG = -1e30

_mesh = plsc.VectorSubcoreMesh(core_axis_name="c", subcore_axis_name="s")


def _agg_body(h_hbm, src_hbm, dst_hbm, meta_hbm, t_hbm, out_hbm,
              stage, rows, srcw, dstw, mrow_v, tv):
    wid = lax.axis_index("c") * 16 + lax.axis_index("s")
    pltpu.sync_copy(meta_hbm.at[wid], mrow_v)
    pltpu.sync_copy(t_hbm, tv)
    mv = mrow_v[...]
    e0 = mv[0]
    e1 = mv[1]
    n0 = mv[2]

    @pl.loop(0, NPT * H, step=16)
    def _(i):
        stage[pl.ds(i, 16)] = jnp.zeros((16,), jnp.float32)

    t_vec = tv[...]

    def make_edge_body(jstart, jend):
        def edge_body(j, carry):
            cur, ms, ss, ns = carry
            valid = jnp.logical_and(j >= jstart, j < jend)
            d = jnp.where(valid, dstw[pl.ds(j, 16)][0], cur)
            changed = d != cur

            @pl.when(jnp.logical_and(changed, cur >= 0))
            def _():
                r = cur - n0
                for k in range(8):
                    stage[pl.ds(r * H + 16 * k, 16)] = ns[k] / ss[k]

            new_m, new_s, new_n = [], [], []
            for k in range(8):
                row = rows[j, pl.ds(16 * k, 16)]
                msg = jnp.maximum(row, 0.0) + EPS
                alpha = jnp.where(valid, msg * t_vec, NEG)
                m_eff = jnp.where(changed, NEG, ms[k])
                ge = alpha >= m_eff
                q = jnp.exp(-jnp.abs(alpha - m_eff))
                scale = jnp.where(ge, q, 1.0)
                w = jnp.where(ge, 1.0, q)
                new_m.append(jnp.maximum(m_eff, alpha))
                new_s.append(ss[k] * scale + w)
                new_n.append(ns[k] * scale + msg * w)
            return (d, tuple(new_m), tuple(new_s), tuple(new_n))
        return edge_body

    def batch_body(bb, carry):
        base = bb * BE
        pltpu.sync_copy(src_hbm.at[pl.ds(base, BE)], srcw)
        pltpu.sync_copy(dst_hbm.at[pl.ds(base, BE)], dstw.at[pl.ds(0, BE)])
        pltpu.sync_copy(h_hbm.at[srcw], rows)
        jstart = jnp.maximum(e0 - base, 0)
        jend = jnp.minimum(e1 - base, BE)
        return lax.fori_loop(0, BE, make_edge_body(jstart, jend), carry,
                             unroll=4)

    init = (jnp.int32(-1),
            tuple(jnp.full((16,), NEG, jnp.float32) for _ in range(8)),
            tuple(jnp.zeros((16,), jnp.float32) for _ in range(8)),
            tuple(jnp.zeros((16,), jnp.float32) for _ in range(8)))
    bb0 = e0 // BE
    bb1 = (e1 + BE - 1) // BE
    cur, ms, ss, ns = lax.fori_loop(bb0, bb1, batch_body, init)

    @pl.when(cur >= 0)
    def _():
        r = cur - n0
        for k in range(8):
            stage[pl.ds(r * H + 16 * k, 16)] = ns[k] / ss[k]

    pltpu.sync_copy(stage, out_hbm.at[pl.ds(n0 * H, NPT * H)])


@jax.jit
def _sc_aggregate(h, src_s, dst_s, meta, t16):
    k = pl.kernel(
        _agg_body,
        out_type=jax.ShapeDtypeStruct((N2 * H,), jnp.float32),
        mesh=_mesh,
        scratch_types=[
            pltpu.VMEM((NPT * H,), jnp.float32),
            pltpu.VMEM((BE, H), jnp.float32),
            pltpu.VMEM((BE,), jnp.int32),
            pltpu.VMEM((BE + 16,), jnp.int32),
            pltpu.VMEM((16,), jnp.int32),
            pltpu.VMEM((16,), jnp.float32),
        ],
    )
    return k(h, src_s, dst_s, meta, t16).reshape(N2, H)


def _linear_body(x_ref, w_ref, b_ref, o_ref):
    o_ref[...] = (
        jnp.dot(x_ref[...], w_ref[...], preferred_element_type=jnp.float32)
        + b_ref[...]
    )


@jax.jit
def _linear(x, w, b):
    return pl.pallas_call(
        _linear_body,
        grid=(N2 // ROW_BLK,),
        in_specs=[
            pl.BlockSpec((ROW_BLK, D), lambda i: (i, 0)),
            pl.BlockSpec((D, H), lambda i: (0, 0)),
            pl.BlockSpec((1, H), lambda i: (0, 0)),
        ],
        out_specs=pl.BlockSpec((ROW_BLK, H), lambda i: (i, 0)),
        out_shape=jax.ShapeDtypeStruct((N2, H), jnp.float32),
    )(x, w, b.reshape(1, H))


def _mlp_body(mode, agg_ref, h_ref, w1_ref, g1_ref, b1_ref, w2_ref, o_ref):
    out = agg_ref[...] + h_ref[...]
    hh = jnp.dot(out, w1_ref[...], preferred_element_type=jnp.float32)
    hh = hh * (1.0 / jnp.sqrt(1.0 + BN_EPS)) * g1_ref[...] + b1_ref[...]
    hh = jnp.maximum(hh, 0.0)
    o = jnp.dot(hh, w2_ref[...], preferred_element_type=jnp.float32)
    if mode == "relu":
        o = jnp.maximum(o, 0.0)
    elif mode == "clamp":
        o = jnp.minimum(o, MAX_LOGSTD)
    o_ref[...] = o


@functools.partial(jax.jit, static_argnames=("mode",))
def _mlp(agg, h, w1, g1, b1, w2, mode):
    return pl.pallas_call(
        functools.partial(_mlp_body, mode),
        grid=(N2 // ROW_BLK,),
        in_specs=[
            pl.BlockSpec((ROW_BLK, H), lambda i: (i, 0)),
            pl.BlockSpec((ROW_BLK, H), lambda i: (i, 0)),
            pl.BlockSpec((H, 2 * H), lambda i: (0, 0)),
            pl.BlockSpec((1, 2 * H), lambda i: (0, 0)),
            pl.BlockSpec((1, 2 * H), lambda i: (0, 0)),
            pl.BlockSpec((2 * H, H), lambda i: (0, 0)),
        ],
        out_specs=pl.BlockSpec((ROW_BLK, H), lambda i: (i, 0)),
        out_shape=jax.ShapeDtypeStruct((N2, H), jnp.float32),
    )(agg, h, w1, g1.reshape(1, 2 * H), b1.reshape(1, 2 * H), w2)


def _reparam_body(mu_ref, ls_ref, nz_ref, o_ref):
    o_ref[...] = mu_ref[...] + nz_ref[...] * jnp.exp(ls_ref[...])


@jax.jit
def _reparam(mu, logstd, noise):
    return pl.pallas_call(
        _reparam_body,
        grid=(N2 // ROW_BLK,),
        in_specs=[pl.BlockSpec((ROW_BLK, H), lambda i: (i, 0))] * 3,
        out_specs=pl.BlockSpec((ROW_BLK, H), lambda i: (i, 0)),
        out_shape=jax.ShapeDtypeStruct((N2, H), jnp.float32),
    )(mu, logstd, noise)


def kernel(x, edge_index, Wn, bn, t, W1, g1, b1, W2, noise):
    src = edge_index[0].astype(jnp.int32)
    dst = edge_index[1].astype(jnp.int32)

    perm = jnp.argsort(dst)
    dst_s = dst[perm]
    src_s = src[perm]
    tile_nodes = jnp.arange(NW + 1, dtype=jnp.int32) * NPT
    bounds = jnp.searchsorted(dst_s, tile_nodes).astype(jnp.int32)
    meta = jnp.concatenate(
        [bounds[:-1, None], bounds[1:, None], tile_nodes[:-1, None],
         jnp.zeros((NW, 13), jnp.int32)], axis=1)

    x_p = jnp.pad(x, ((0, N2 - N), (0, 0)))
    noise_p = jnp.pad(noise, ((0, N2 - N), (0, 0)))

    def agg(h, i):
        return _sc_aggregate(h, src_s, dst_s, meta, jnp.full((16,), t[i]))

    h = _linear(x_p, Wn, bn)
    for i in range(2):
        h = _mlp(agg(h, i), h, W1[i], g1[i], b1[i], W2[i], "relu")
    agg2 = agg(h, 2)
    mu = _mlp(agg2, h, W1[2], g1[2], b1[2], W2[2], "none")
    agg3 = lax.cond(t[2] == t[3], lambda: agg2, lambda: agg(h, 3))
    logstd = _mlp(agg3, h, W1[3], g1[3], b1[3], W2[3], "clamp")
    zh = Z = _reparam(mu, logstd, noise_p)
    for i in (4, 5):
        zh = _mlp(agg(zh, i), zh, W1[i], g1[i], b1[i], W2[i], "relu")
    return (zh[:N], Z[:N])


# speedup vs baseline: 7.6005x; 1.1066x over previous
import functools

import jax
import jax.numpy as jnp
from jax import lax
from jax.experimental import pallas as pl
from jax.experimental.pallas import tpu as pltpu
from jax.experimental.pallas import tpu_sc as plsc

N = 10000
E = 320000
D = 128
H = 128
EPS = 1e-7
MAX_LOGSTD = 10.0
BN_EPS = 1e-5

NW = 32
NPT = 313
N2 = NW * NPT
BE = 128
ROW_BLK = 2504
NEG = -1e30

_mesh = plsc.VectorSubcoreMesh(core_axis_name="c", subcore_axis_name="s")


def _agg_body(h_hbm, src_hbm, dst_hbm, meta_hbm, t_hbm, out_hbm,
              stage, rows, srcw, dstw, mrow_v, tv):
    wid = lax.axis_index("c") * 16 + lax.axis_index("s")
    pltpu.sync_copy(meta_hbm.at[wid], mrow_v)
    pltpu.sync_copy(t_hbm, tv)
    mv = mrow_v[...]
    e0 = mv[0]
    e1 = mv[1]
    n0 = mv[2]

    @pl.loop(0, NPT * H, step=16)
    def _(i):
        stage[pl.ds(i, 16)] = jnp.zeros((16,), jnp.float32)

    t_vec = tv[...]

    def edge_body(j, carry):
        cur, ms, ss, ns = carry
        d = dstw[pl.ds(j, 16)][0]
        changed = d != cur

        @pl.when(jnp.logical_and(changed, cur >= 0))
        def _():
            r = cur - n0
            for k in range(8):
                stage[pl.ds(r * H + 16 * k, 16)] = ns[k] / ss[k]

        new_m, new_s, new_n = [], [], []
        for k in range(8):
            row = rows[j, pl.ds(16 * k, 16)]
            msg = jnp.maximum(row, 0.0) + EPS
            alpha = msg * t_vec
            m_eff = jnp.where(changed, NEG, ms[k])
            ge = alpha >= m_eff
            q = jnp.exp(-jnp.abs(alpha - m_eff))
            scale = jnp.where(ge, q, 1.0)
            w = jnp.where(ge, 1.0, q)
            new_m.append(jnp.maximum(m_eff, alpha))
            new_s.append(ss[k] * scale + w)
            new_n.append(ns[k] * scale + msg * w)
        return (d, tuple(new_m), tuple(new_s), tuple(new_n))

    def batch_body(bb, carry):
        base = bb * BE
        pltpu.sync_copy(src_hbm.at[pl.ds(base, BE)], srcw)
        pltpu.sync_copy(dst_hbm.at[pl.ds(base, BE)], dstw.at[pl.ds(0, BE)])
        pltpu.sync_copy(h_hbm.at[srcw], rows)
        jstart = jnp.maximum(e0 - base, 0)
        jend = jnp.minimum(e1 - base, BE)
        return lax.fori_loop(jstart, jend, edge_body, carry)

    init = (jnp.int32(-1),
            tuple(jnp.full((16,), NEG, jnp.float32) for _ in range(8)),
            tuple(jnp.zeros((16,), jnp.float32) for _ in range(8)),
            tuple(jnp.zeros((16,), jnp.float32) for _ in range(8)))
    bb0 = e0 // BE
    bb1 = (e1 + BE - 1) // BE
    cur, ms, ss, ns = lax.fori_loop(bb0, bb1, batch_body, init)

    @pl.when(cur >= 0)
    def _():
        r = cur - n0
        for k in range(8):
            stage[pl.ds(r * H + 16 * k, 16)] = ns[k] / ss[k]

    pltpu.sync_copy(stage, out_hbm.at[pl.ds(n0 * H, NPT * H)])


@jax.jit
def _sc_aggregate(h, src_s, dst_s, meta, t16):
    k = pl.kernel(
        _agg_body,
        out_type=jax.ShapeDtypeStruct((N2 * H,), jnp.float32),
        mesh=_mesh,
        scratch_types=[
            pltpu.VMEM((NPT * H,), jnp.float32),
            pltpu.VMEM((BE, H), jnp.float32),
            pltpu.VMEM((BE,), jnp.int32),
            pltpu.VMEM((BE + 16,), jnp.int32),
            pltpu.VMEM((16,), jnp.int32),
            pltpu.VMEM((16,), jnp.float32),
        ],
    )
    return k(h, src_s, dst_s, meta, t16).reshape(N2, H)


def _linear_body(x_ref, w_ref, b_ref, o_ref):
    o_ref[...] = (
        jnp.dot(x_ref[...], w_ref[...], preferred_element_type=jnp.float32)
        + b_ref[...]
    )


@jax.jit
def _linear(x, w, b):
    return pl.pallas_call(
        _linear_body,
        grid=(N2 // ROW_BLK,),
        in_specs=[
            pl.BlockSpec((ROW_BLK, D), lambda i: (i, 0)),
            pl.BlockSpec((D, H), lambda i: (0, 0)),
            pl.BlockSpec((1, H), lambda i: (0, 0)),
        ],
        out_specs=pl.BlockSpec((ROW_BLK, H), lambda i: (i, 0)),
        out_shape=jax.ShapeDtypeStruct((N2, H), jnp.float32),
    )(x, w, b.reshape(1, H))


def _mlp_body(mode, agg_ref, h_ref, w1_ref, g1_ref, b1_ref, w2_ref, o_ref):
    out = agg_ref[...] + h_ref[...]
    hh = jnp.dot(out, w1_ref[...], preferred_element_type=jnp.float32)
    hh = hh * (1.0 / jnp.sqrt(1.0 + BN_EPS)) * g1_ref[...] + b1_ref[...]
    hh = jnp.maximum(hh, 0.0)
    o = jnp.dot(hh, w2_ref[...], preferred_element_type=jnp.float32)
    if mode == "relu":
        o = jnp.maximum(o, 0.0)
    elif mode == "clamp":
        o = jnp.minimum(o, MAX_LOGSTD)
    o_ref[...] = o


@functools.partial(jax.jit, static_argnames=("mode",))
def _mlp(agg, h, w1, g1, b1, w2, mode):
    return pl.pallas_call(
        functools.partial(_mlp_body, mode),
        grid=(N2 // ROW_BLK,),
        in_specs=[
            pl.BlockSpec((ROW_BLK, H), lambda i: (i, 0)),
            pl.BlockSpec((ROW_BLK, H), lambda i: (i, 0)),
            pl.BlockSpec((H, 2 * H), lambda i: (0, 0)),
            pl.BlockSpec((1, 2 * H), lambda i: (0, 0)),
            pl.BlockSpec((1, 2 * H), lambda i: (0, 0)),
            pl.BlockSpec((2 * H, H), lambda i: (0, 0)),
        ],
        out_specs=pl.BlockSpec((ROW_BLK, H), lambda i: (i, 0)),
        out_shape=jax.ShapeDtypeStruct((N2, H), jnp.float32),
    )(agg, h, w1, g1.reshape(1, 2 * H), b1.reshape(1, 2 * H), w2)


def _reparam_body(mu_ref, ls_ref, nz_ref, o_ref):
    o_ref[...] = mu_ref[...] + nz_ref[...] * jnp.exp(ls_ref[...])


@jax.jit
def _reparam(mu, logstd, noise):
    return pl.pallas_call(
        _reparam_body,
        grid=(N2 // ROW_BLK,),
        in_specs=[pl.BlockSpec((ROW_BLK, H), lambda i: (i, 0))] * 3,
        out_specs=pl.BlockSpec((ROW_BLK, H), lambda i: (i, 0)),
        out_shape=jax.ShapeDtypeStruct((N2, H), jnp.float32),
    )(mu, logstd, noise)


def kernel(x, edge_index, Wn, bn, t, W1, g1, b1, W2, noise):
    src = edge_index[0].astype(jnp.int32)
    dst = edge_index[1].astype(jnp.int32)

    perm = jnp.argsort(dst)
    dst_s = dst[perm]
    src_s = src[perm]
    tile_nodes = jnp.arange(NW + 1, dtype=jnp.int32) * NPT
    bounds = jnp.searchsorted(dst_s, tile_nodes).astype(jnp.int32)
    meta = jnp.concatenate(
        [bounds[:-1, None], bounds[1:, None], tile_nodes[:-1, None],
         jnp.zeros((NW, 13), jnp.int32)], axis=1)

    x_p = jnp.pad(x, ((0, N2 - N), (0, 0)))
    noise_p = jnp.pad(noise, ((0, N2 - N), (0, 0)))

    def agg(h, i):
        return _sc_aggregate(h, src_s, dst_s, meta, jnp.full((16,), t[i]))

    h = _linear(x_p, Wn, bn)
    for i in range(2):
        h = _mlp(agg(h, i), h, W1[i], g1[i], b1[i], W2[i], "relu")
    agg2 = agg(h, 2)
    mu = _mlp(agg2, h, W1[2], g1[2], b1[2], W2[2], "none")
    agg3 = lax.cond(t[2] == t[3], lambda: agg2, lambda: agg(h, 3))
    logstd = _mlp(agg3, h, W1[3], g1[3], b1[3], W2[3], "clamp")
    zh = Z = _reparam(mu, logstd, noise_p)
    for i in (4, 5):
        zh = _mlp(agg(zh, i), zh, W1[i], g1[i], b1[i], W2[i], "relu")
    return (zh[:N], Z[:N])
